# Initial kernel scaffold; baseline (speedup 1.0000x reference)
import functools

import jax
import jax.numpy as jnp
from jax import lax
from jax.experimental import pallas as pl
from jax.experimental.pallas import tpu as pltpu
from jax.experimental.pallas import tpu_sc as plsc

VOCAB = 1000
NC, NS = 2, 16
NW = NC * NS
N = 1024 * 50
ROWS_PER_W = N // NW
CS = 50
NCHUNK = ROWS_PER_W // CS


@functools.partial(
    pl.kernel,
    out_type=jax.ShapeDtypeStruct((N, VOCAB), jnp.float32),
    mesh=plsc.VectorSubcoreMesh(core_axis_name="c", subcore_axis_name="s"),
    scratch_types=[
        pltpu.VMEM((NCHUNK, CS), jnp.int32),
        pltpu.VMEM((CS, VOCAB), jnp.float32),
        pltpu.SemaphoreType.DMA,
    ],
)
def _gather_rows(table_hbm, idx_hbm, out_hbm, idx_v, buf, sem):
    wid = lax.axis_index("s") * NC + lax.axis_index("c")
    pltpu.sync_copy(idx_hbm.at[wid], idx_v)

    def step(j, carry):
        pltpu.async_copy(table_hbm.at[idx_v.at[j]], buf, sem).wait()
        pltpu.sync_copy(buf, out_hbm.at[pl.ds(wid * ROWS_PER_W + j * CS, CS)])
        return carry

    lax.fori_loop(0, NCHUNK, step, 0)


def kernel(idx, targets, token_embedding_table):
    B, T = idx.shape
    idx3 = idx.astype(jnp.int32).reshape(NW, NCHUNK, CS)
    out = _gather_rows(token_embedding_table, idx3)
    return out.reshape(B, T, VOCAB)


# speedup vs baseline: 1.0149x; 1.0149x over previous
import functools

import jax
import jax.numpy as jnp
from jax import lax
from jax.experimental import pallas as pl
from jax.experimental.pallas import tpu as pltpu
from jax.experimental.pallas import tpu_sc as plsc

VOCAB = 1000
NC, NS = 2, 16
NW = NC * NS
N = 1024 * 50
ROWS_PER_W = N // NW
CS = 64
NCHUNK = ROWS_PER_W // CS


@functools.partial(
    pl.kernel,
    out_type=jax.ShapeDtypeStruct((N, VOCAB), jnp.float32),
    mesh=plsc.VectorSubcoreMesh(core_axis_name="c", subcore_axis_name="s"),
    scratch_types=[
        pltpu.VMEM((NCHUNK, CS), jnp.int32),
        pltpu.VMEM((CS, VOCAB), jnp.float32),
        pltpu.SemaphoreType.DMA,
    ],
    compiler_params=pltpu.CompilerParams(use_tc_tiling_on_sc=False),
)
def _gather_rows(table_hbm, idx_hbm, out_hbm, idx_v, buf, sem):
    wid = lax.axis_index("s") * NC + lax.axis_index("c")
    pltpu.sync_copy(idx_hbm.at[wid], idx_v)

    def step(j, carry):
        pltpu.async_copy(table_hbm.at[idx_v.at[j]], buf, sem).wait()
        pltpu.sync_copy(buf, out_hbm.at[pl.ds(wid * ROWS_PER_W + j * CS, CS)])
        return carry

    lax.fori_loop(0, NCHUNK, step, 0)


def kernel(idx, targets, token_embedding_table):
    B, T = idx.shape
    idx3 = idx.astype(jnp.int32).reshape(NW, NCHUNK, CS)
    out = _gather_rows(token_embedding_table, idx3)
    return out.reshape(B, T, VOCAB)


# speedup vs baseline: 1.0173x; 1.0024x over previous
import functools

import jax
import jax.numpy as jnp
from jax import lax
from jax.experimental import pallas as pl
from jax.experimental.pallas import tpu as pltpu
from jax.experimental.pallas import tpu_sc as plsc

VOCAB = 1000
NC, NS = 2, 16
NW = NC * NS
N = 1024 * 50
ROWS_PER_W = N // NW
CS = 40
NCHUNK = ROWS_PER_W // CS
NPAIR = NCHUNK // 2


@functools.partial(
    pl.kernel,
    out_type=jax.ShapeDtypeStruct((N, VOCAB), jnp.float32),
    mesh=plsc.VectorSubcoreMesh(core_axis_name="c", subcore_axis_name="s"),
    scratch_types=[
        pltpu.VMEM((NCHUNK, CS), jnp.int32),
        pltpu.VMEM((CS, VOCAB), jnp.float32),
        pltpu.VMEM((CS, VOCAB), jnp.float32),
        pltpu.SemaphoreType.DMA,
        pltpu.SemaphoreType.DMA,
        pltpu.SemaphoreType.DMA,
        pltpu.SemaphoreType.DMA,
    ],
    compiler_params=pltpu.CompilerParams(use_tc_tiling_on_sc=False),
)
def _gather_rows(table_hbm, idx_hbm, out_hbm, idx_v, b0, b1, gs0, gs1, ss0, ss1):
    wid = lax.axis_index("s") * NC + lax.axis_index("c")
    base = wid * ROWS_PER_W
    pltpu.sync_copy(idx_hbm.at[wid], idx_v)

    def gather(j, buf, sem):
        return pltpu.async_copy(table_hbm.at[idx_v.at[j]], buf, sem)

    def scatter(j, buf, sem):
        return pltpu.async_copy(buf, out_hbm.at[pl.ds(base + j * CS, CS)], sem)

    gather(0, b0, gs0)
    gather(1, b1, gs1)

    def pair(i, carry):
        j = 2 * i
        pltpu.make_async_copy(table_hbm.at[idx_v.at[j]], b0, gs0).wait()
        scatter(j, b0, ss0)
        pltpu.make_async_copy(table_hbm.at[idx_v.at[j + 1]], b1, gs1).wait()
        scatter(j + 1, b1, ss1)

        @pl.when(i < NPAIR - 1)
        def _():
            pltpu.make_async_copy(b0, out_hbm.at[pl.ds(base + j * CS, CS)], ss0).wait()
            gather(j + 2, b0, gs0)
            pltpu.make_async_copy(b1, out_hbm.at[pl.ds(base + (j + 1) * CS, CS)], ss1).wait()
            gather(j + 3, b1, gs1)

        return carry

    lax.fori_loop(0, NPAIR, pair, 0)
    jlast = NCHUNK - 2
    pltpu.make_async_copy(b0, out_hbm.at[pl.ds(base + jlast * CS, CS)], ss0).wait()
    pltpu.make_async_copy(b1, out_hbm.at[pl.ds(base + (jlast + 1) * CS, CS)], ss1).wait()


def kernel(idx, targets, token_embedding_table):
    B, T = idx.shape
    idx3 = idx.astype(jnp.int32).reshape(NW, NCHUNK, CS)
    out = _gather_rows(token_embedding_table, idx3)
    return out.reshape(B, T, VOCAB)
